# Initial kernel scaffold; baseline (speedup 1.0000x reference)
import jax
import jax.numpy as jnp
from jax.experimental import pallas as pl


def kernel(h, edge_index, W1_0, b1_0, W2_0, b2_0, W1_1, b1_1, W2_1, b2_1, W1_2, b1_2, W2_2, b2_2):
    raise NotImplementedError("write your pallas kernel here")


# speedup vs baseline: 9.4274x; 9.4274x over previous
import functools

import jax
import jax.numpy as jnp
from jax import lax
from jax.experimental import pallas as pl
from jax.experimental.pallas import tpu as pltpu
from jax.experimental.pallas import tpu_sc as plsc

_N = 10000
_E = 320000
_DH = 32
_NP = 10112

_NC = 2
_NS = 16
_NW = _NC * _NS
_BATCH = 128
_K = 80
_EPAD = _NW * _BATCH * _K
_RPT = _NP // _NS


_mesh = plsc.VectorSubcoreMesh(core_axis_name="c", subcore_axis_name="s")


@functools.partial(
    pl.kernel,
    out_type=jax.ShapeDtypeStruct((_NC * _NP, _DH), jnp.float32),
    mesh=_mesh,
    scratch_types=[
        pltpu.VMEM((_K, _BATCH), jnp.int32),
        pltpu.VMEM((_K, _BATCH), jnp.int32),
        pltpu.VMEM((2, _BATCH, _DH), jnp.float32),
        pltpu.VMEM_SHARED((_NP, _DH), jnp.float32),
        pltpu.SemaphoreType.DMA,
        pltpu.SemaphoreType.DMA,
    ],
    compiler_params=pltpu.CompilerParams(use_tc_tiling_on_sc=False),
)
def _sc_agg(y_hbm, src_hbm, dst_hbm, zeros_hbm, out_hbm,
            src_v, dst_v, rows_v, agg_sh, sem0, sem1):
    cid = lax.axis_index("c")
    sid = lax.axis_index("s")
    wid = cid * _NS + sid
    rbase = sid * _RPT

    pltpu.sync_copy(zeros_hbm, agg_sh.at[pl.ds(rbase, _RPT)])
    pltpu.sync_copy(src_hbm.at[wid], src_v)
    pltpu.sync_copy(dst_hbm.at[wid], dst_v)
    plsc.subcore_barrier()

    sems = (sem0, sem1)
    pltpu.async_copy(y_hbm.at[src_v.at[0]], rows_v.at[0], sems[0])

    def body(kk, _):
        k0 = kk * 2
        for b in (0, 1):
            k = k0 + b

            @pl.when(k + 1 < _K)
            def _():
                pltpu.async_copy(y_hbm.at[src_v.at[k + 1]], rows_v.at[1 - b],
                                 sems[1 - b])

            pltpu.make_async_copy(y_hbm.at[src_v.at[k]], rows_v.at[b],
                                  sems[b]).wait()
            pltpu.sync_copy(rows_v.at[b], agg_sh.at[dst_v.at[k]], add=True)
        return 0

    lax.fori_loop(0, _K // 2, body, 0)
    plsc.subcore_barrier()

    pltpu.sync_copy(agg_sh.at[pl.ds(rbase, _RPT)],
                    out_hbm.at[pl.ds(cid * _NP + rbase, _RPT)])


def _tc_in_body(h_ref, w1_ref, y_ref):
    y_ref[...] = jnp.dot(h_ref[...], w1_ref[...],
                         preferred_element_type=jnp.float32)


def _tc_mid_body(y_ref, agg_ref, b1_ref, w2_ref, b2_ref, w1n_ref,
                 yn_ref, s_ref):
    z = y_ref[...] + agg_ref[0] + agg_ref[1] + b1_ref[...]
    z = jnp.maximum(z, 0.0)
    xn = jnp.dot(z, w2_ref[...], preferred_element_type=jnp.float32)
    xn = xn + b2_ref[...]
    mask = lax.broadcasted_iota(jnp.int32, xn.shape, 0) < _N
    xn = jnp.where(mask, xn, 0.0)
    yn_ref[...] = jnp.dot(xn, w1n_ref[...], preferred_element_type=jnp.float32)
    s_ref[...] = jnp.sum(xn, axis=0, keepdims=True) * (1.0 / _N)


def _tc_out_body(y_ref, agg_ref, b1_ref, w2_ref, b2_ref, s_ref):
    z = y_ref[...] + agg_ref[0] + agg_ref[1] + b1_ref[...]
    z = jnp.maximum(z, 0.0)
    xn = jnp.dot(z, w2_ref[...], preferred_element_type=jnp.float32)
    xn = xn + b2_ref[...]
    mask = lax.broadcasted_iota(jnp.int32, xn.shape, 0) < _N
    xn = jnp.where(mask, xn, 0.0)
    s_ref[...] = jnp.sum(xn, axis=0, keepdims=True) * (1.0 / _N)


_tc_in = pl.pallas_call(
    _tc_in_body,
    out_shape=jax.ShapeDtypeStruct((_NP, _DH), jnp.float32),
)

_tc_mid = pl.pallas_call(
    _tc_mid_body,
    out_shape=(
        jax.ShapeDtypeStruct((_NP, _DH), jnp.float32),
        jax.ShapeDtypeStruct((1, _DH), jnp.float32),
    ),
)

_tc_out = pl.pallas_call(
    _tc_out_body,
    out_shape=jax.ShapeDtypeStruct((1, _DH), jnp.float32),
)


def kernel(h, edge_index, W1_0, b1_0, W2_0, b2_0, W1_1, b1_1, W2_1, b2_1,
           W1_2, b1_2, W2_2, b2_2):
    src = edge_index[0]
    dst = edge_index[1]
    pad = _EPAD - _E
    srcp = jnp.concatenate(
        [src, jnp.full((pad,), _N, jnp.int32)]).reshape(_NW, _K, _BATCH)
    dstp = jnp.concatenate(
        [dst, jnp.zeros((pad,), jnp.int32)]).reshape(_NW, _K, _BATCH)
    h_pad = jnp.pad(h, ((0, _NP - _N), (0, 0)))
    zeros = jnp.zeros((_RPT, _DH), jnp.float32)

    y = _tc_in(h_pad, W1_0)

    agg = _sc_agg(y, srcp, dstp, zeros).reshape(_NC, _NP, _DH)
    y, s0 = _tc_mid(y, agg, b1_0.reshape(1, _DH), W2_0,
                    b2_0.reshape(1, _DH), W1_1)

    agg = _sc_agg(y, srcp, dstp, zeros).reshape(_NC, _NP, _DH)
    y, s1 = _tc_mid(y, agg, b1_1.reshape(1, _DH), W2_1,
                    b2_1.reshape(1, _DH), W1_2)

    agg = _sc_agg(y, srcp, dstp, zeros).reshape(_NC, _NP, _DH)
    s2 = _tc_out(y, agg, b1_2.reshape(1, _DH), W2_2, b2_2.reshape(1, _DH))

    return jnp.concatenate([s0[0], s1[0], s2[0]])
